# Initial kernel scaffold; baseline (speedup 1.0000x reference)
import jax
import jax.numpy as jnp
from jax.experimental import pallas as pl


def kernel(rpn_boxes, rpn_scores, img_height, img_width):
    raise NotImplementedError("write your pallas kernel here")


# speedup vs baseline: 112.2936x; 112.2936x over previous
import functools

import jax
import jax.numpy as jnp
from jax import lax
from jax.experimental import pallas as pl
from jax.experimental.pallas import tpu as pltpu

_THRESH = 0.7
_N = 5000
_NP = 5120
_B = 512
_C = _NP // _B
_OUT = 1000
_OPAD = 1024


def _nms_body(rows_ref, cols_ref, full_ref, out_ref,
              keep_ref, alive_ref, arr_ref, cum_ref):
    r = pl.program_id(0)

    cxr = rows_ref[:, 0:1]
    cyr = rows_ref[:, 1:2]
    wr = rows_ref[:, 2:3]
    hr = rows_ref[:, 3:4]
    x1r = cxr - 0.5 * wr
    y1r = cyr - 0.5 * hr
    x2r = cxr + 0.5 * wr
    y2r = cyr + 0.5 * hr
    area_r = (x2r - x1r) * (y2r - y1r)
    rowg = r * _B + lax.broadcasted_iota(jnp.int32, (_B, 1), 0)

    @pl.when(r == 0)
    def _init():
        for c in range(_C):
            colg0 = c * _B + lax.broadcasted_iota(jnp.int32, (1, _B), 1)
            keep_ref[c] = (colg0 < _N).astype(jnp.float32)

    def compute_a(c):
        col = cols_ref[c]
        cxc = col[0:1, :]
        cyc = col[1:2, :]
        wc = col[2:3, :]
        hc = col[3:4, :]
        x1c = cxc - 0.5 * wc
        y1c = cyc - 0.5 * hc
        x2c = cxc + 0.5 * wc
        y2c = cyc + 0.5 * hc
        area_c = (x2c - x1c) * (y2c - y1c)
        xx1 = jnp.maximum(x1r, x1c)
        yy1 = jnp.maximum(y1r, y1c)
        xx2 = jnp.minimum(x2r, x2c)
        yy2 = jnp.minimum(y2r, y2c)
        iw = jnp.maximum(xx2 - xx1, 0.0)
        ih = jnp.maximum(yy2 - yy1, 0.0)
        inter = iw * ih
        union = area_r + area_c - inter
        iou = inter / jnp.maximum(union, 1e-9)
        colg = c * _B + lax.broadcasted_iota(jnp.int32, (1, _B), 1)
        a = (iou > _THRESH) & (colg > rowg)
        return a.astype(jnp.bfloat16)

    dot11 = functools.partial(
        lax.dot_general,
        dimension_numbers=(((1,), (0,)), ((), ())),
        preferred_element_type=jnp.float32,
    )

    arr_ref[...] = compute_a(r)
    init = keep_ref[r]
    alive_ref[...] = init

    def w_body(_):
        old = alive_ref[...]
        sup = dot11(old.astype(jnp.bfloat16), arr_ref[...])
        new = jnp.where((sup == 0.0) & (init > 0.0), 1.0, 0.0)
        alive_ref[...] = new
        return jnp.any(new != old)

    lax.while_loop(lambda ch: ch, w_body, True)
    alive = alive_ref[...]
    keep_ref[r] = alive
    alive_b = alive.astype(jnp.bfloat16)

    def cross(c, carry):
        a_c = compute_a(c)
        sup = dot11(alive_b, a_c)
        keep_ref[c] = keep_ref[c] * (sup == 0.0).astype(jnp.float32)
        return carry

    lax.fori_loop(r + 1, _C, cross, 0)

    @pl.when(r == _C - 1)
    def _select():
        ii = lax.broadcasted_iota(jnp.int32, (_B, _B), 0)
        jj = lax.broadcasted_iota(jnp.int32, (_B, _B), 1)
        tri = (ii <= jj).astype(jnp.bfloat16)

        def cum_body(c, off):
            k = keep_ref[c]
            cc = dot11(k.astype(jnp.bfloat16), tri) + off
            cum_ref[c] = cc
            return cc[0:1, _B - 1:_B]

        total = lax.fori_loop(0, _C, cum_body, jnp.zeros((1, 1), jnp.float32))

        oo = lax.broadcasted_iota(jnp.int32, (_OPAD, 1), 0).astype(jnp.float32)
        lane0 = lax.broadcasted_iota(jnp.int32, (1, _B), 1) == 0

        def sel_body(c, acc):
            cc = cum_ref[c]
            k = keep_ref[c]
            onehot = (k > 0.0) & (cc == oo + 1.0)
            fill = (oo + 1.0 > total) & lane0 & (c == 0)
            p = (onehot | fill).astype(jnp.float32)
            boxes_c = full_ref[pl.ds(c * _B, _B), :]
            return acc + lax.dot_general(
                p, boxes_c,
                dimension_numbers=(((1,), (0,)), ((), ())),
                precision=lax.Precision.HIGHEST,
                preferred_element_type=jnp.float32,
            )

        acc = lax.fori_loop(0, _C, sel_body, jnp.zeros((_OPAD, 4), jnp.float32))
        out_ref[...] = acc[0:_OUT, :]


def _nms_pallas(boxes_padded, cols, interpret=False):
    return pl.pallas_call(
        _nms_body,
        grid=(_C,),
        in_specs=[
            pl.BlockSpec((_B, 4), lambda r: (r, 0)),
            pl.BlockSpec((_C, 4, _B), lambda r: (0, 0, 0)),
            pl.BlockSpec((_NP, 4), lambda r: (0, 0)),
        ],
        out_specs=pl.BlockSpec((_OUT, 4), lambda r: (0, 0)),
        out_shape=jax.ShapeDtypeStruct((_OUT, 4), jnp.float32),
        scratch_shapes=[
            pltpu.VMEM((_C, 1, _B), jnp.float32),
            pltpu.VMEM((1, _B), jnp.float32),
            pltpu.VMEM((_B, _B), jnp.bfloat16),
            pltpu.VMEM((_C, 1, _B), jnp.float32),
        ],
        interpret=interpret,
    )(boxes_padded, cols, boxes_padded)


def kernel(rpn_boxes, rpn_scores, img_height, img_width):
    scores = jax.nn.sigmoid(rpn_scores.squeeze(1))
    order = jnp.argsort(-scores)
    boxes_sorted = rpn_boxes[order]
    boxes_padded = jnp.pad(boxes_sorted, ((0, _NP - _N), (0, 0)))
    cols = boxes_padded.T.reshape(4, _C, _B).transpose(1, 0, 2)
    return _nms_pallas(boxes_padded, cols)


# speedup vs baseline: 122.5249x; 1.0911x over previous
import functools

import jax
import jax.numpy as jnp
from jax import lax
from jax.experimental import pallas as pl
from jax.experimental.pallas import tpu as pltpu

_THRESH = 0.7
_N = 5000
_NP = 5120
_B = 512
_C = _NP // _B
_OUT = 1000
_OPAD = 1024


def _nms_body(rows_ref, cols_ref, full_ref, out_ref,
              keep_ref, alive_ref, arr_ref, cum_ref):
    r = pl.program_id(0)

    cxr = rows_ref[:, 0:1]
    cyr = rows_ref[:, 1:2]
    wr = rows_ref[:, 2:3]
    hr = rows_ref[:, 3:4]
    x1r = cxr - 0.5 * wr
    y1r = cyr - 0.5 * hr
    x2r = cxr + 0.5 * wr
    y2r = cyr + 0.5 * hr
    area_r = (x2r - x1r) * (y2r - y1r)
    rowg = r * _B + lax.broadcasted_iota(jnp.int32, (_B, 1), 0)

    @pl.when(r == 0)
    def _init():
        for c in range(_C):
            colg0 = c * _B + lax.broadcasted_iota(jnp.int32, (1, _B), 1)
            keep_ref[c] = (colg0 < _N).astype(jnp.float32)

    def compute_a(c, diag):
        col = cols_ref[c]
        cxc = col[0:1, :]
        cyc = col[1:2, :]
        wc = col[2:3, :]
        hc = col[3:4, :]
        x1c = cxc - 0.5 * wc
        y1c = cyc - 0.5 * hc
        x2c = cxc + 0.5 * wc
        y2c = cyc + 0.5 * hc
        area_c = (x2c - x1c) * (y2c - y1c)
        xx1 = jnp.maximum(x1r, x1c)
        yy1 = jnp.maximum(y1r, y1c)
        xx2 = jnp.minimum(x2r, x2c)
        yy2 = jnp.minimum(y2r, y2c)
        iw = jnp.maximum(xx2 - xx1, 0.0)
        ih = jnp.maximum(yy2 - yy1, 0.0)
        inter = iw * ih
        union = area_r + area_c - inter
        iou = inter / union
        a = iou > _THRESH
        if diag:
            colg = c * _B + lax.broadcasted_iota(jnp.int32, (1, _B), 1)
            a = a & (colg > rowg)
        return a.astype(jnp.bfloat16)

    dot11 = functools.partial(
        lax.dot_general,
        dimension_numbers=(((1,), (0,)), ((), ())),
        preferred_element_type=jnp.float32,
    )

    arr_ref[...] = compute_a(r, diag=True)
    init = keep_ref[r]
    alive_ref[...] = init

    def w_body(_):
        old = alive_ref[...]
        sup = dot11(old.astype(jnp.bfloat16), arr_ref[...])
        new = jnp.where((sup == 0.0) & (init > 0.0), 1.0, 0.0)
        alive_ref[...] = new
        return jnp.any(new != old)

    lax.while_loop(lambda ch: ch, w_body, True)
    alive = alive_ref[...]
    keep_ref[r] = alive
    alive_b = alive.astype(jnp.bfloat16)

    def cross(c, carry):
        a_c = compute_a(c, diag=False)
        sup = dot11(alive_b, a_c)
        keep_ref[c] = keep_ref[c] * (sup == 0.0).astype(jnp.float32)
        return carry

    lax.fori_loop(r + 1, _C, cross, 0)

    @pl.when(r == _C - 1)
    def _select():
        ii = lax.broadcasted_iota(jnp.int32, (_B, _B), 0)
        jj = lax.broadcasted_iota(jnp.int32, (_B, _B), 1)
        tri = (ii <= jj).astype(jnp.bfloat16)

        def cum_body(c, off):
            k = keep_ref[c]
            cc = dot11(k.astype(jnp.bfloat16), tri) + off
            cum_ref[c] = cc
            return cc[0:1, _B - 1:_B]

        total = lax.fori_loop(0, _C, cum_body, jnp.zeros((1, 1), jnp.float32))

        oo = lax.broadcasted_iota(jnp.int32, (_OPAD, 1), 0).astype(jnp.float32)
        lane0 = lax.broadcasted_iota(jnp.int32, (1, _B), 1) == 0

        def sel_body(c, acc):
            cc = cum_ref[c]
            k = keep_ref[c]
            onehot = (k > 0.0) & (cc == oo + 1.0)
            fill = (oo + 1.0 > total) & lane0 & (c == 0)
            p = (onehot | fill).astype(jnp.bfloat16)
            boxes_c = full_ref[pl.ds(c * _B, _B), :]
            hi = boxes_c.astype(jnp.bfloat16)
            r1 = boxes_c - hi.astype(jnp.float32)
            mid = r1.astype(jnp.bfloat16)
            lo = (r1 - mid.astype(jnp.float32)).astype(jnp.bfloat16)
            for part in (hi, mid, lo):
                acc = acc + dot11(p, part)
            return acc

        acc = lax.fori_loop(0, _C, sel_body, jnp.zeros((_OPAD, 4), jnp.float32))
        out_ref[...] = acc[0:_OUT, :]


def _nms_pallas(boxes_padded, cols, interpret=False):
    return pl.pallas_call(
        _nms_body,
        grid=(_C,),
        in_specs=[
            pl.BlockSpec((_B, 4), lambda r: (r, 0)),
            pl.BlockSpec((_C, 4, _B), lambda r: (0, 0, 0)),
            pl.BlockSpec((_NP, 4), lambda r: (0, 0)),
        ],
        out_specs=pl.BlockSpec((_OUT, 4), lambda r: (0, 0)),
        out_shape=jax.ShapeDtypeStruct((_OUT, 4), jnp.float32),
        scratch_shapes=[
            pltpu.VMEM((_C, 1, _B), jnp.float32),
            pltpu.VMEM((1, _B), jnp.float32),
            pltpu.VMEM((_B, _B), jnp.bfloat16),
            pltpu.VMEM((_C, 1, _B), jnp.float32),
        ],
        interpret=interpret,
    )(boxes_padded, cols, boxes_padded)


def kernel(rpn_boxes, rpn_scores, img_height, img_width):
    scores = jax.nn.sigmoid(rpn_scores.squeeze(1))
    order = jnp.argsort(-scores)
    boxes_sorted = rpn_boxes[order]
    boxes_padded = jnp.pad(boxes_sorted, ((0, _NP - _N), (0, 0)))
    del img_height, img_width
    boxes_padded = jnp.pad(boxes_sorted, ((0, _NP - _N), (0, 0)))
    cols = boxes_padded.T.reshape(4, _C, _B).transpose(1, 0, 2)
    return _nms_pallas(boxes_padded, cols)


# speedup vs baseline: 127.6717x; 1.0420x over previous
import functools

import jax
import jax.numpy as jnp
from jax import lax
from jax.experimental import pallas as pl
from jax.experimental.pallas import tpu as pltpu

_THRESH = 0.7
_N = 5000
_NP = 5120
_B = 1024
_C = _NP // _B
_OUT = 1000
_OPAD = 1024


def _nms_body(rows_ref, cols_ref, full_ref, out_ref,
              keep_ref, alive_ref, arr_ref, cum_ref):
    r = pl.program_id(0)

    cxr = rows_ref[:, 0:1]
    cyr = rows_ref[:, 1:2]
    wr = rows_ref[:, 2:3]
    hr = rows_ref[:, 3:4]
    x1r = cxr - 0.5 * wr
    y1r = cyr - 0.5 * hr
    x2r = cxr + 0.5 * wr
    y2r = cyr + 0.5 * hr
    area_r = (x2r - x1r) * (y2r - y1r)
    rowg = r * _B + lax.broadcasted_iota(jnp.int32, (_B, 1), 0)

    @pl.when(r == 0)
    def _init():
        for c in range(_C):
            colg0 = c * _B + lax.broadcasted_iota(jnp.int32, (1, _B), 1)
            keep_ref[c] = (colg0 < _N).astype(jnp.float32)

    def compute_a(c, diag):
        col = cols_ref[c]
        cxc = col[0:1, :]
        cyc = col[1:2, :]
        wc = col[2:3, :]
        hc = col[3:4, :]
        x1c = cxc - 0.5 * wc
        y1c = cyc - 0.5 * hc
        x2c = cxc + 0.5 * wc
        y2c = cyc + 0.5 * hc
        area_c = (x2c - x1c) * (y2c - y1c)
        xx1 = jnp.maximum(x1r, x1c)
        yy1 = jnp.maximum(y1r, y1c)
        xx2 = jnp.minimum(x2r, x2c)
        yy2 = jnp.minimum(y2r, y2c)
        iw = jnp.maximum(xx2 - xx1, 0.0)
        ih = jnp.maximum(yy2 - yy1, 0.0)
        inter = iw * ih
        union = area_r + area_c - inter
        iou = inter / union
        a = iou > _THRESH
        if diag:
            colg = c * _B + lax.broadcasted_iota(jnp.int32, (1, _B), 1)
            a = a & (colg > rowg)
        return a.astype(jnp.bfloat16)

    dot11 = functools.partial(
        lax.dot_general,
        dimension_numbers=(((1,), (0,)), ((), ())),
        preferred_element_type=jnp.float32,
    )

    arr_ref[...] = compute_a(r, diag=True)
    init = keep_ref[r]
    alive_ref[...] = init

    def w_body(_):
        old = alive_ref[...]
        sup = dot11(old.astype(jnp.bfloat16), arr_ref[...])
        new = jnp.where((sup == 0.0) & (init > 0.0), 1.0, 0.0)
        alive_ref[...] = new
        return jnp.any(new != old)

    lax.while_loop(lambda ch: ch, w_body, True)
    alive = alive_ref[...]
    keep_ref[r] = alive
    alive_b = alive.astype(jnp.bfloat16)

    def cross(c, carry):
        a_c = compute_a(c, diag=False)
        sup = dot11(alive_b, a_c)
        keep_ref[c] = keep_ref[c] * (sup == 0.0).astype(jnp.float32)
        return carry

    lax.fori_loop(r + 1, _C, cross, 0)

    @pl.when(r == _C - 1)
    def _select():
        ii = lax.broadcasted_iota(jnp.int32, (_B, _B), 0)
        jj = lax.broadcasted_iota(jnp.int32, (_B, _B), 1)
        tri = (ii <= jj).astype(jnp.bfloat16)

        def cum_body(c, off):
            k = keep_ref[c]
            cc = dot11(k.astype(jnp.bfloat16), tri) + off
            cum_ref[c] = cc
            return cc[0:1, _B - 1:_B]

        total = lax.fori_loop(0, _C, cum_body, jnp.zeros((1, 1), jnp.float32))

        oo = lax.broadcasted_iota(jnp.int32, (_OPAD, 1), 0).astype(jnp.float32)
        lane0 = lax.broadcasted_iota(jnp.int32, (1, _B), 1) == 0

        def sel_body(c, acc):
            cc = cum_ref[c]
            k = keep_ref[c]
            onehot = (k > 0.0) & (cc == oo + 1.0)
            fill = (oo + 1.0 > total) & lane0 & (c == 0)
            p = (onehot | fill).astype(jnp.bfloat16)
            boxes_c = full_ref[pl.ds(c * _B, _B), :]
            hi = boxes_c.astype(jnp.bfloat16)
            r1 = boxes_c - hi.astype(jnp.float32)
            mid = r1.astype(jnp.bfloat16)
            lo = (r1 - mid.astype(jnp.float32)).astype(jnp.bfloat16)
            for part in (hi, mid, lo):
                acc = acc + dot11(p, part)
            return acc

        acc = lax.fori_loop(0, _C, sel_body, jnp.zeros((_OPAD, 4), jnp.float32))
        out_ref[...] = acc[0:_OUT, :]


def _nms_pallas(boxes_padded, cols, interpret=False):
    return pl.pallas_call(
        _nms_body,
        grid=(_C,),
        in_specs=[
            pl.BlockSpec((_B, 4), lambda r: (r, 0)),
            pl.BlockSpec((_C, 4, _B), lambda r: (0, 0, 0)),
            pl.BlockSpec((_NP, 4), lambda r: (0, 0)),
        ],
        out_specs=pl.BlockSpec((_OUT, 4), lambda r: (0, 0)),
        out_shape=jax.ShapeDtypeStruct((_OUT, 4), jnp.float32),
        scratch_shapes=[
            pltpu.VMEM((_C, 1, _B), jnp.float32),
            pltpu.VMEM((1, _B), jnp.float32),
            pltpu.VMEM((_B, _B), jnp.bfloat16),
            pltpu.VMEM((_C, 1, _B), jnp.float32),
        ],
        interpret=interpret,
    )(boxes_padded, cols, boxes_padded)


def kernel(rpn_boxes, rpn_scores, img_height, img_width):
    scores = jax.nn.sigmoid(rpn_scores.squeeze(1))
    order = jnp.argsort(-scores)
    boxes_sorted = rpn_boxes[order]
    boxes_padded = jnp.pad(boxes_sorted, ((0, _NP - _N), (0, 0)))
    del img_height, img_width
    boxes_padded = jnp.pad(boxes_sorted, ((0, _NP - _N), (0, 0)))
    cols = boxes_padded.T.reshape(4, _C, _B).transpose(1, 0, 2)
    return _nms_pallas(boxes_padded, cols)


# speedup vs baseline: 136.8664x; 1.0720x over previous
import functools

import jax
import jax.numpy as jnp
from jax import lax
from jax.experimental import pallas as pl
from jax.experimental.pallas import tpu as pltpu

_THRESH = 0.7
_N = 5000
_NP = 5120
_B = 1024
_C = _NP // _B
_OUT = 1000
_OPAD = 1024


def _split3(x):
    hi = x.astype(jnp.bfloat16)
    r1 = x - hi.astype(jnp.float32)
    mid = r1.astype(jnp.bfloat16)
    lo = (r1 - mid.astype(jnp.float32)).astype(jnp.bfloat16)
    return hi, mid, lo


def _nms_body(raw_t_ref, ord_ref, out_ref,
              keep_ref, alive_ref, arr_ref, cum_ref, rows_ref, cols_ref):
    r = pl.program_id(0)

    dot11 = functools.partial(
        lax.dot_general,
        dimension_numbers=(((1,), (0,)), ((), ())),
        preferred_element_type=jnp.float32,
    )

    @pl.when(r == 0)
    def _init():
        parts = _split3(raw_t_ref[...])
        jglob = lax.broadcasted_iota(jnp.int32, (_NP, 1), 0)
        for c in range(_C):
            oc = ord_ref[c]
            q = (oc == jglob).astype(jnp.bfloat16)
            col4 = (dot11(parts[0], q) + dot11(parts[1], q)
                    + dot11(parts[2], q))
            cols_ref[c] = col4
            rows_ref[c * _B:(c + 1) * _B, :] = col4.T
            colg0 = c * _B + lax.broadcasted_iota(jnp.int32, (1, _B), 1)
            keep_ref[c] = (colg0 < _N).astype(jnp.float32)

    cxr = rows_ref[pl.ds(r * _B, _B), 0:1]
    cyr = rows_ref[pl.ds(r * _B, _B), 1:2]
    wr = rows_ref[pl.ds(r * _B, _B), 2:3]
    hr = rows_ref[pl.ds(r * _B, _B), 3:4]
    x1r = cxr - 0.5 * wr
    y1r = cyr - 0.5 * hr
    x2r = cxr + 0.5 * wr
    y2r = cyr + 0.5 * hr
    area_r = (x2r - x1r) * (y2r - y1r)
    rowg = r * _B + lax.broadcasted_iota(jnp.int32, (_B, 1), 0)

    def compute_a(c, diag):
        col = cols_ref[c]
        cxc = col[0:1, :]
        cyc = col[1:2, :]
        wc = col[2:3, :]
        hc = col[3:4, :]
        x1c = cxc - 0.5 * wc
        y1c = cyc - 0.5 * hc
        x2c = cxc + 0.5 * wc
        y2c = cyc + 0.5 * hc
        area_c = (x2c - x1c) * (y2c - y1c)
        xx1 = jnp.maximum(x1r, x1c)
        yy1 = jnp.maximum(y1r, y1c)
        xx2 = jnp.minimum(x2r, x2c)
        yy2 = jnp.minimum(y2r, y2c)
        iw = jnp.maximum(xx2 - xx1, 0.0)
        ih = jnp.maximum(yy2 - yy1, 0.0)
        inter = iw * ih
        union = area_r + area_c - inter
        iou = inter / union
        a = iou > _THRESH
        if diag:
            colg = c * _B + lax.broadcasted_iota(jnp.int32, (1, _B), 1)
            a = a & (colg > rowg)
        return a.astype(jnp.bfloat16)

    arr_ref[...] = compute_a(r, diag=True)
    init = keep_ref[r]
    alive_ref[...] = init

    def w_body(_):
        old = alive_ref[...]
        sup = dot11(old.astype(jnp.bfloat16), arr_ref[...])
        new = jnp.where((sup == 0.0) & (init > 0.0), 1.0, 0.0)
        alive_ref[...] = new
        return jnp.any(new != old)

    lax.while_loop(lambda ch: ch, w_body, True)
    alive = alive_ref[...]
    keep_ref[r] = alive
    alive_b = alive.astype(jnp.bfloat16)

    def cross(c, carry):
        a_c = compute_a(c, diag=False)
        sup = dot11(alive_b, a_c)
        keep_ref[c] = keep_ref[c] * (sup == 0.0).astype(jnp.float32)
        return carry

    lax.fori_loop(r + 1, _C, cross, 0)

    @pl.when(r == _C - 1)
    def _select():
        ii = lax.broadcasted_iota(jnp.int32, (_B, _B), 0)
        jj = lax.broadcasted_iota(jnp.int32, (_B, _B), 1)
        tri = (ii <= jj).astype(jnp.bfloat16)

        def cum_body(c, off):
            k = keep_ref[c]
            cc = dot11(k.astype(jnp.bfloat16), tri) + off
            cum_ref[c] = cc
            return cc[0:1, _B - 1:_B]

        total = lax.fori_loop(0, _C, cum_body, jnp.zeros((1, 1), jnp.float32))

        oo = lax.broadcasted_iota(jnp.int32, (_OPAD, 1), 0).astype(jnp.float32)
        lane0 = lax.broadcasted_iota(jnp.int32, (1, _B), 1) == 0

        def sel_body(c, acc):
            cc = cum_ref[c]
            k = keep_ref[c]
            onehot = (k > 0.0) & (cc == oo + 1.0)
            fill = (oo + 1.0 > total) & lane0 & (c == 0)
            p = (onehot | fill).astype(jnp.bfloat16)
            boxes_c = rows_ref[pl.ds(c * _B, _B), :]
            for part in _split3(boxes_c):
                acc = acc + dot11(p, part)
            return acc

        acc = lax.fori_loop(0, _C, sel_body, jnp.zeros((_OPAD, 4), jnp.float32))
        out_ref[...] = acc[0:_OUT, :]


def _nms_pallas(raw_t, order_chunks, interpret=False):
    return pl.pallas_call(
        _nms_body,
        grid=(_C,),
        in_specs=[
            pl.BlockSpec((4, _NP), lambda r: (0, 0)),
            pl.BlockSpec((_C, 1, _B), lambda r: (0, 0, 0)),
        ],
        out_specs=pl.BlockSpec((_OUT, 4), lambda r: (0, 0)),
        out_shape=jax.ShapeDtypeStruct((_OUT, 4), jnp.float32),
        scratch_shapes=[
            pltpu.VMEM((_C, 1, _B), jnp.float32),
            pltpu.VMEM((1, _B), jnp.float32),
            pltpu.VMEM((_B, _B), jnp.bfloat16),
            pltpu.VMEM((_C, 1, _B), jnp.float32),
            pltpu.VMEM((_NP, 4), jnp.float32),
            pltpu.VMEM((_C, 4, _B), jnp.float32),
        ],
        interpret=interpret,
    )(raw_t, order_chunks)


def _run(rpn_boxes, rpn_scores, interpret=False):
    scores = jax.nn.sigmoid(rpn_scores.squeeze(1))
    order = jnp.argsort(-scores)
    order_pad = jnp.concatenate([order.astype(jnp.int32),
                                 jnp.arange(_N, _NP, dtype=jnp.int32)])
    order_chunks = order_pad.reshape(_C, 1, _B)
    raw_t = jnp.pad(rpn_boxes, ((0, _NP - _N), (0, 0))).T
    return _nms_pallas(raw_t, order_chunks, interpret=interpret)


def kernel(rpn_boxes, rpn_scores, img_height, img_width):
    del img_height, img_width
    return _run(rpn_boxes, rpn_scores)
